# baseline (speedup 1.0000x reference)
import functools

import jax
import jax.numpy as jnp
from jax import lax
from jax.experimental import pallas as pl
from jax.experimental.pallas import tpu as pltpu
from jax.experimental.pallas import tpu_sc as plsc

_NUM_CORES = 2
_NUM_SUBCORES = 16
_NW = _NUM_CORES * _NUM_SUBCORES
_LANES = 16
_CHUNK = 256


@functools.lru_cache(maxsize=None)
def _make_gather(n_planes: int, n_b: int, d: int):
    b_per_w = n_b // _NW
    halves = b_per_w // _CHUNK
    n_chunks = n_planes * halves
    assert b_per_w % _CHUNK == 0 and d % _LANES == 0 and halves == 2
    mesh = plsc.VectorSubcoreMesh(core_axis_name="c", subcore_axis_name="s")

    @functools.partial(
        pl.kernel,
        out_type=jax.ShapeDtypeStruct((n_planes, d, n_b), jnp.float32),
        mesh=mesh,
        scratch_types=[
            pltpu.VMEM((n_planes, b_per_w), jnp.int32),
            pltpu.VMEM((2, _CHUNK, d), jnp.float32),
            pltpu.VMEM((2, d, _CHUNK), jnp.float32),
            pltpu.SemaphoreType.DMA((2,)),
            pltpu.SemaphoreType.DMA((2,)),
        ],
        compiler_params=pltpu.CompilerParams(
            use_tc_tiling_on_sc=False, needs_layout_passes=False,
            disable_bounds_checks=True),
    )
    def gather_kernel(idx_hbm, table_hbm, out_hbm, idx_v, gbuf, tbuf, gsem, ssem):
        wid = lax.axis_index("s") * _NUM_CORES + lax.axis_index("c")
        pltpu.sync_copy(idx_hbm.at[:, wid], idx_v)

        def gather_chunk(t, half, k):
            return pltpu.make_async_copy(
                table_hbm.at[idx_v.at[t, pl.ds(half * _CHUNK, _CHUNK)]],
                gbuf.at[k], gsem.at[k])

        def store_chunk(t, half, k):
            b0 = wid * b_per_w + half * _CHUNK
            return pltpu.make_async_copy(
                tbuf.at[k], out_hbm.at[t, :, pl.ds(b0, _CHUNK)], ssem.at[k])

        def transpose_chunk(k):
            src, dst = gbuf.at[k], tbuf.at[k]

            def col(c, carry):
                cols = jnp.full((_LANES,), c, dtype=jnp.int32)
                lane = jnp.arange(_LANES, dtype=jnp.int32)
                for g in range(_CHUNK // _LANES):
                    vals = plsc.load_gather(src, [lane + g * _LANES, cols])
                    dst[c, pl.ds(g * _LANES, _LANES)] = vals
                return carry

            lax.fori_loop(0, d, col, 0)

        gather_chunk(0, 0, 0).start()

        def body(t, carry):
            for half in range(2):
                k = half
                u = t * 2 + half
                gather_chunk(t, half, k).wait()
                pl.when(u >= 2)(lambda: store_chunk(t - 1, half, k).wait())
                transpose_chunk(k)
                tn = t + (half + 1) // 2
                hn = (half + 1) % 2
                pl.when(u + 1 < n_chunks)(
                    lambda: gather_chunk(tn, hn, 1 - k).start())
                store_chunk(t, half, k).start()
            return carry

        lax.fori_loop(0, n_planes, body, 0)
        store_chunk(n_planes - 1, 0, 0).wait()
        store_chunk(n_planes - 1, 1, 1).wait()

    return gather_kernel


def kernel(token_ids, weight):
    b, t = token_ids.shape
    d = weight.shape[1]
    idx3 = jnp.transpose(token_ids).astype(jnp.int32).reshape(t, _NW, b // _NW)
    out = _make_gather(t, b, d)(idx3, weight)
    return jnp.transpose(out, (2, 0, 1))


# speedup vs baseline: 1.7319x; 1.7319x over previous
import functools

import jax
import jax.numpy as jnp
from jax import lax
from jax.experimental import pallas as pl
from jax.experimental.pallas import tpu as pltpu
from jax.experimental.pallas import tpu_sc as plsc

_NUM_CORES = 2
_NUM_SUBCORES = 16
_NW = _NUM_CORES * _NUM_SUBCORES
_CHUNK = 512
_NBUF = 2
_LOOKAHEAD = 1


@functools.lru_cache(maxsize=None)
def _make_gather(b_total: int, d: int):
    chunk = _CHUNK
    assert b_total % (_NW * chunk) == 0
    b_per_w = b_total // _NW
    n_chunks = b_per_w // chunk
    assert n_chunks % _NBUF == 0
    mesh = plsc.VectorSubcoreMesh(core_axis_name="c", subcore_axis_name="s")

    @functools.partial(
        pl.kernel,
        out_type=jax.ShapeDtypeStruct((b_total, d), jnp.float32),
        mesh=mesh,
        scratch_types=[
            pltpu.VMEM((n_chunks, chunk), jnp.int32),
            pltpu.VMEM((_NBUF, chunk, d), jnp.float32),
            pltpu.SemaphoreType.DMA((_NBUF,)),
            pltpu.SemaphoreType.DMA((_NBUF,)),
        ],
        compiler_params=pltpu.CompilerParams(use_tc_tiling_on_sc=False),
    )
    def gather_kernel(idx_hbm, table_hbm, out_hbm, idx_v, rows_v, gsem, ssem):
        wid = lax.axis_index("s") * _NUM_CORES + lax.axis_index("c")
        pltpu.sync_copy(idx_hbm.at[wid], idx_v)

        def gather_chunk(i, b):
            return pltpu.make_async_copy(
                table_hbm.at[idx_v.at[i]], rows_v.at[b], gsem.at[b])

        def store_chunk(i, b):
            return pltpu.make_async_copy(
                rows_v.at[b],
                out_hbm.at[pl.ds(wid * b_per_w + i * chunk, chunk)],
                ssem.at[b])

        for i0 in range(_LOOKAHEAD):
            gather_chunk(i0, i0 % _NBUF).start()

        def body(j, carry):
            for b in range(_NBUF):
                i = j * _NBUF + b
                b2 = (b + _LOOKAHEAD) % _NBUF
                gather_chunk(i, b).wait()
                store_chunk(i, b).start()
                pl.when(i >= _NBUF - _LOOKAHEAD)(
                    lambda: store_chunk(i + _LOOKAHEAD - _NBUF, b2).wait())
                pl.when(i + _LOOKAHEAD < n_chunks)(
                    lambda: gather_chunk(i + _LOOKAHEAD, b2).start())
            return carry

        lax.fori_loop(0, n_chunks // _NBUF, body, 0)
        for i0 in range(n_chunks - (_NBUF - _LOOKAHEAD), n_chunks):
            store_chunk(i0, i0 % _NBUF).wait()

    return gather_kernel


def kernel(token_ids, weight):
    b, t = token_ids.shape
    d = weight.shape[1]
    idx = token_ids.astype(jnp.int32).reshape(_NW, -1, _CHUNK)
    out = _make_gather(b * t, d)(idx, weight)
    return out.reshape(b, t, d)
